# baseline (speedup 1.0000x reference)
import functools

import jax
import jax.numpy as jnp
from jax import lax
from jax.experimental import pallas as pl
from jax.experimental.pallas import tpu as pltpu
from jax.experimental.pallas import tpu_sc as plsc

N = 10000
E = 320000
D = 128
H = 128
A = 64
G = 64
C = 3

NC = 2
NS = 16
NW = NC * NS
EPW = E // NW
JB = 8

C2 = 125
NCH2 = EPW // C2
NJB2 = NCH2 // JB

C4 = 25
CP4 = 32
NCH4 = EPW // C4
NJB4 = NCH4 // JB

NP = 10240
TPS = NP // NS
BN = 2000
NB = N // BN


def _round_bf16(v):
    u = lax.bitcast_convert_type(v, jnp.int32)
    lsb = lax.shift_right_logical(u, 16) & 1
    r = u + 32767 + lsb
    r = r & jnp.int32(-65536)
    return lax.bitcast_convert_type(r, jnp.float32)


def _sc_mesh():
    return plsc.VectorSubcoreMesh(
        core_axis_name="c", subcore_axis_name="s",
        num_cores=NC, num_subcores=NS)


def _stage1_body(x_ref, W1_ref, b1_ref, Wc1_ref, bc1_ref, h_ref, g_ref):
    xb = x_ref[...]
    h_ref[...] = jnp.maximum(
        jnp.dot(xb, W1_ref[...], preferred_element_type=jnp.float32)
        + b1_ref[...], 0.0)
    g_ref[...] = jnp.maximum(
        jnp.dot(xb, Wc1_ref[...], preferred_element_type=jnp.float32)
        + bc1_ref[...], 0.0)


def _stage1(x, W1, b1, Wc1, bc1):
    return pl.pallas_call(
        _stage1_body,
        grid=(NB,),
        in_specs=[
            pl.BlockSpec((BN, D), lambda i: (i, 0)),
            pl.BlockSpec((D, H), lambda i: (0, 0)),
            pl.BlockSpec((1, H), lambda i: (0, 0)),
            pl.BlockSpec((D, H), lambda i: (0, 0)),
            pl.BlockSpec((1, H), lambda i: (0, 0)),
        ],
        out_specs=[
            pl.BlockSpec((BN, H), lambda i: (i, 0)),
            pl.BlockSpec((BN, H), lambda i: (i, 0)),
        ],
        out_shape=[
            jax.ShapeDtypeStruct((N, H), jnp.float32),
            jax.ShapeDtypeStruct((N, H), jnp.float32),
        ],
    )(x, W1, b1.reshape(1, H), Wc1, bc1.reshape(1, H))


def _segsum_sc(h, src3, dst3, zeros):
    @functools.partial(
        pl.kernel,
        out_type=jax.ShapeDtypeStruct((NC, NP, H), jnp.float32),
        mesh=_sc_mesh(),
        compiler_params=pltpu.CompilerParams(needs_layout_passes=False),
        scratch_types=[
            pltpu.VMEM((JB, C2), jnp.int32),
            pltpu.VMEM((JB, C2), jnp.int32),
            pltpu.VMEM((C2, H), jnp.float32),
            pltpu.VMEM((C2, H), jnp.float32),
            pltpu.VMEM_SHARED((NP, H), jnp.float32),
            pltpu.SemaphoreType.DMA,
            pltpu.SemaphoreType.DMA,
        ],
    )
    def k(h_hbm, src_hbm, dst_hbm, z_hbm, out_hbm, srcb, dstb, rows0, rows1,
          acc_sh, sem0, sem1):
        c = lax.axis_index("c")
        s = lax.axis_index("s")
        wid = c * NS + s
        pltpu.sync_copy(z_hbm.at[pl.ds(s * TPS, TPS)],
                        acc_sh.at[pl.ds(s * TPS, TPS)])
        plsc.subcore_barrier()
        bufs = (rows0, rows1)
        sems = (sem0, sem1)

        def blk(jb, carry):
            base = jb * JB
            pltpu.sync_copy(src_hbm.at[wid, pl.ds(base, JB)], srcb)
            pltpu.sync_copy(dst_hbm.at[wid, pl.ds(base, JB)], dstb)
            pltpu.async_copy(h_hbm.at[srcb.at[0]], rows0, sem0)
            for jj in range(JB):
                b = bufs[jj % 2]
                sm = sems[jj % 2]
                pltpu.make_async_copy(h_hbm.at[srcb.at[jj]], b, sm).wait()
                if jj + 1 < JB:
                    pltpu.async_copy(h_hbm.at[srcb.at[jj + 1]],
                                     bufs[(jj + 1) % 2], sems[(jj + 1) % 2])
                pltpu.sync_copy(b, acc_sh.at[dstb.at[jj]], add=True)
            return carry

        lax.fori_loop(0, NJB2, blk, 0)
        plsc.subcore_barrier()
        pltpu.sync_copy(acc_sh.at[pl.ds(s * TPS, TPS)],
                        out_hbm.at[c, pl.ds(s * TPS, TPS)])

    return k(h, src3, dst3, zeros)


def _stage3_body(h_ref, m0_ref, m1_ref, g_ref, W2_ref, b2_ref, Was_ref,
                 Wad_ref, ba1_ref, st_ref, qt_ref):
    emb = jnp.maximum(
        jnp.dot(h_ref[...] + m0_ref[...] + m1_ref[...], W2_ref[...],
                preferred_element_type=jnp.float32) + b2_ref[...], 0.0)
    p = (jnp.dot(emb, Was_ref[...], preferred_element_type=jnp.float32)
         + ba1_ref[...])
    q = jnp.dot(emb, Wad_ref[...], preferred_element_type=jnp.float32)
    pad = jnp.zeros((BN, H - A), jnp.float32)
    st_ref[...] = jnp.concatenate([g_ref[...], p, pad], axis=1)
    qt_ref[...] = jnp.concatenate([q, pad], axis=1)


def _stage3(h, m0, m1, g0, W2, b2, Was, Wad, ba1):
    return pl.pallas_call(
        _stage3_body,
        grid=(NB,),
        in_specs=[
            pl.BlockSpec((BN, H), lambda i: (i, 0)),
            pl.BlockSpec((BN, H), lambda i: (i, 0)),
            pl.BlockSpec((BN, H), lambda i: (i, 0)),
            pl.BlockSpec((BN, H), lambda i: (i, 0)),
            pl.BlockSpec((H, H), lambda i: (0, 0)),
            pl.BlockSpec((1, H), lambda i: (0, 0)),
            pl.BlockSpec((H, A), lambda i: (0, 0)),
            pl.BlockSpec((H, A), lambda i: (0, 0)),
            pl.BlockSpec((1, A), lambda i: (0, 0)),
        ],
        out_specs=[
            pl.BlockSpec((BN, 2 * H), lambda i: (i, 0)),
            pl.BlockSpec((BN, H), lambda i: (i, 0)),
        ],
        out_shape=[
            jax.ShapeDtypeStruct((N, 2 * H), jnp.float32),
            jax.ShapeDtypeStruct((N, H), jnp.float32),
        ],
    )(h, m0, m1, g0, W2, b2.reshape(1, H), Was, Wad, ba1.reshape(1, A))


def _edge_att_sc(st, qt, src3, dst3, zeros, wx):
    @functools.partial(
        pl.kernel,
        out_type=[
            jax.ShapeDtypeStruct((NW, NCH4, H), jnp.float32),
            jax.ShapeDtypeStruct((NC, NP, H), jnp.float32),
        ],
        mesh=_sc_mesh(),
        compiler_params=pltpu.CompilerParams(needs_layout_passes=False),
        scratch_types=[
            pltpu.VMEM((JB, CP4), jnp.int32),
            pltpu.VMEM((JB, CP4), jnp.int32),
            pltpu.VMEM((CP4, 2 * H), jnp.float32),
            pltpu.VMEM((CP4, 2 * H), jnp.float32),
            pltpu.VMEM((CP4, H), jnp.float32),
            pltpu.VMEM((CP4, H), jnp.float32),
            pltpu.VMEM((CP4, H), jnp.float32),
            pltpu.VMEM((JB, H), jnp.float32),
            pltpu.VMEM((8, 16), jnp.float32),
            pltpu.VMEM_SHARED((NP, H), jnp.float32),
            pltpu.SemaphoreType.DMA,
            pltpu.SemaphoreType.DMA,
            pltpu.SemaphoreType.DMA,
            pltpu.SemaphoreType.DMA,
        ],
    )
    def k(st_hbm, qt_hbm, src_hbm, dst_hbm, z_hbm, wx_hbm,
          att_hbm, m2_hbm,
          srcb, dstb, sv0, sv1, qt0, qt1, attg, attb, wx_v, acc_sh,
          ssem0, ssem1, qsem0, qsem1):
        c = lax.axis_index("c")
        s = lax.axis_index("s")
        wid = c * NS + s
        pltpu.sync_copy(z_hbm.at[pl.ds(s * TPS, TPS)],
                        acc_sh.at[pl.ds(s * TPS, TPS)])
        pltpu.sync_copy(wx_hbm, wx_v)
        plsc.subcore_barrier()

        svs = (sv0, sv1)
        qts = (qt0, qt1)
        ssems = (ssem0, ssem1)
        qsems = (qsem0, qsem1)

        def start(jj, ph):
            pltpu.async_copy(st_hbm.at[srcb.at[jj]], svs[ph], ssems[ph])
            pltpu.async_copy(qt_hbm.at[dstb.at[jj]], qts[ph], qsems[ph])

        def wait(jj, ph):
            pltpu.make_async_copy(st_hbm.at[srcb.at[jj]],
                                  svs[ph], ssems[ph]).wait()
            pltpu.make_async_copy(qt_hbm.at[dstb.at[jj]],
                                  qts[ph], qsems[ph]).wait()

        def compute(jj, ph):
            s_v = svs[ph]
            qt_v = qts[ph]
            w0 = wx_v[0]
            w1 = wx_v[1]
            w2 = wx_v[2]
            w3 = wx_v[3]

            def grp(gi, cc2):
                acc = wx_v[4]
                for l in range(16):
                    e = gi * 16 + l
                    a0 = jnp.maximum(
                        s_v[e, pl.ds(H, 16)] + qt_v[e, pl.ds(0, 16)], 0.0)
                    a1 = jnp.maximum(
                        s_v[e, pl.ds(H + 16, 16)] + qt_v[e, pl.ds(16, 16)],
                        0.0)
                    a2 = jnp.maximum(
                        s_v[e, pl.ds(H + 32, 16)] + qt_v[e, pl.ds(32, 16)],
                        0.0)
                    a3 = jnp.maximum(
                        s_v[e, pl.ds(H + 48, 16)] + qt_v[e, pl.ds(48, 16)],
                        0.0)
                    t = (_round_bf16(a0) * w0 + _round_bf16(a1) * w1
                         + _round_bf16(a2) * w2 + _round_bf16(a3) * w3)
                    sv = jnp.sum(t)
                    acc = jnp.where(lax.iota(jnp.int32, 16) == l, sv, acc)
                attvec = 1.0 / (1.0 + jnp.exp(-acc))
                attvec = jnp.where(
                    lax.iota(jnp.int32, 16) < C4 - gi * 16, attvec, 0.0)
                attb[jj, pl.ds(gi * 16, 16)] = attvec
                for l in range(16):
                    av = attvec[l]
                    e = gi * 16 + l
                    for i in range(H // 16):
                        attg[e, pl.ds(i * 16, 16)] = (
                            av * s_v[e, pl.ds(i * 16, 16)])
                return cc2

            lax.fori_loop(0, CP4 // 16, grp, 0)
            pltpu.sync_copy(attg, acc_sh.at[dstb.at[jj]], add=True)

        def blk(jb, carry):
            base = jb * JB
            pltpu.sync_copy(src_hbm.at[wid, pl.ds(base, JB)], srcb)
            pltpu.sync_copy(dst_hbm.at[wid, pl.ds(base, JB)], dstb)
            start(0, 0)
            for jj in range(JB):
                wait(jj, jj % 2)
                if jj + 1 < JB:
                    start(jj + 1, (jj + 1) % 2)
                compute(jj, jj % 2)
            pltpu.sync_copy(attb, att_hbm.at[wid, pl.ds(base, JB)])
            return carry

        lax.fori_loop(0, NJB4, blk, 0)
        plsc.subcore_barrier()
        pltpu.sync_copy(acc_sh.at[pl.ds(s * TPS, TPS)],
                        m2_hbm.at[c, pl.ds(s * TPS, TPS)])

    return k(st, qt, src3, dst3, zeros, wx)


def _stage5_body(g_ref, m0_ref, m1_ref, Wc2_ref, bc2_ref, batch_ref, Wo_ref,
                 bo_ref, out_ref, pool_acc, cnt_acc):
    i = pl.program_id(0)
    gf = jnp.maximum(
        jnp.dot(g_ref[...] + m0_ref[...] + m1_ref[...], Wc2_ref[...],
                preferred_element_type=jnp.float32) + bc2_ref[...], 0.0)
    b = batch_ref[0, 0, :]
    onehot = (b[:, None] == lax.broadcasted_iota(jnp.int32, (BN, G), 1)
              ).astype(jnp.float32)
    pool_p = lax.dot_general(onehot, gf, (((0,), (0,)), ((), ())),
                             preferred_element_type=jnp.float32,
                             precision=lax.Precision.HIGHEST)
    cnt_p = lax.dot_general(onehot, jnp.ones((BN, 1), jnp.float32),
                            (((0,), (0,)), ((), ())),
                            preferred_element_type=jnp.float32,
                            precision=lax.Precision.HIGHEST)

    @pl.when(i == 0)
    def _():
        pool_acc[...] = jnp.zeros_like(pool_acc)
        cnt_acc[...] = jnp.zeros_like(cnt_acc)

    pool_acc[...] += pool_p
    cnt_acc[...] += cnt_p

    @pl.when(i == NB - 1)
    def _():
        pooled = pool_acc[...] / jnp.maximum(cnt_acc[...], 1.0)
        out_ref[...] = (jnp.dot(pooled, Wo_ref[...],
                                preferred_element_type=jnp.float32)
                        + bo_ref[...])


def _stage5(g0, m0, m1, Wc2, bc2, batch3, Wo, bo):
    return pl.pallas_call(
        _stage5_body,
        grid=(NB,),
        in_specs=[
            pl.BlockSpec((BN, H), lambda i: (i, 0)),
            pl.BlockSpec((BN, H), lambda i: (i, 0)),
            pl.BlockSpec((BN, H), lambda i: (i, 0)),
            pl.BlockSpec((H, H), lambda i: (0, 0)),
            pl.BlockSpec((1, H), lambda i: (0, 0)),
            pl.BlockSpec((1, 1, BN), lambda i: (i, 0, 0)),
            pl.BlockSpec((H, C), lambda i: (0, 0)),
            pl.BlockSpec((1, C), lambda i: (0, 0)),
        ],
        out_specs=pl.BlockSpec((G, C), lambda i: (0, 0)),
        out_shape=jax.ShapeDtypeStruct((G, C), jnp.float32),
        scratch_shapes=[
            pltpu.VMEM((G, H), jnp.float32),
            pltpu.VMEM((G, 1), jnp.float32),
        ],
        compiler_params=pltpu.CompilerParams(
            dimension_semantics=("arbitrary",)),
    )(g0, m0, m1, Wc2, bc2.reshape(1, H), batch3, Wo, bo.reshape(1, C))


def kernel(x, W1, b1, W2, b2, Wa1, ba1, Wa2, ba2, Wc1, bc1, Wc2, bc2, Wo, bo,
           edge_index, batch):
    src2 = edge_index[0].reshape(NW, NCH2, C2)
    dst2 = edge_index[1].reshape(NW, NCH2, C2)
    pad4 = jnp.zeros((NW, NCH4, CP4 - C4), jnp.int32)
    src4 = jnp.concatenate(
        [edge_index[0].reshape(NW, NCH4, C4), pad4], axis=2)
    dst4 = jnp.concatenate(
        [edge_index[1].reshape(NW, NCH4, C4), pad4], axis=2)
    zeros = jnp.zeros((NP, H), jnp.float32)
    wx = jnp.zeros((8, 16), jnp.float32)
    wa2_b = Wa2[:, 0].astype(jnp.bfloat16).astype(jnp.float32)
    wx = wx.at[:4].set(wa2_b.reshape(4, 16))
    wx = wx.at[4].set(ba2[0])

    h, g0 = _stage1(x, W1, b1, Wc1, bc1)
    msum = _segsum_sc(h, src2, dst2, zeros)
    st, qt = _stage3(h, msum[0, :N], msum[1, :N], g0, W2, b2,
                     Wa1[:H], Wa1[H:], ba1)
    att3, m2 = _edge_att_sc(st, qt, src4, dst4, zeros, wx)
    logits = _stage5(g0, m2[0, :N], m2[1, :N], Wc2, bc2,
                     batch.reshape(NB, 1, BN), Wo, bo)
    return att3[:, :, :C4].reshape(-1), logits


# speedup vs baseline: 5.1362x; 5.1362x over previous
import functools

import jax
import jax.numpy as jnp
from jax import lax
from jax.experimental import pallas as pl
from jax.experimental.pallas import tpu as pltpu
from jax.experimental.pallas import tpu_sc as plsc

N = 10000
E = 320000
D = 128
H = 128
A = 64
G = 64
C = 3

NC = 2
NS = 16
NW = NC * NS
EPW = E // NW
JB = 8

C2 = 125
NCH2 = EPW // C2
NJB2 = NCH2 // JB

C4 = 25
CP4 = 32
NCH4 = EPW // C4
NJB4 = NCH4 // JB

NP = 10240
TPS = NP // NS
BN = 2000
NB = N // BN


def _round_bf16(v):
    u = lax.bitcast_convert_type(v, jnp.int32)
    lsb = lax.shift_right_logical(u, 16) & 1
    r = u + 32767 + lsb
    r = r & jnp.int32(-65536)
    return lax.bitcast_convert_type(r, jnp.float32)


def _sc_mesh():
    return plsc.VectorSubcoreMesh(
        core_axis_name="c", subcore_axis_name="s",
        num_cores=NC, num_subcores=NS)


def _stage1_body(x_ref, W1_ref, b1_ref, Wc1_ref, bc1_ref, h_ref, g_ref):
    xb = x_ref[...]
    h_ref[...] = jnp.maximum(
        jnp.dot(xb, W1_ref[...], preferred_element_type=jnp.float32)
        + b1_ref[...], 0.0)
    g_ref[...] = jnp.maximum(
        jnp.dot(xb, Wc1_ref[...], preferred_element_type=jnp.float32)
        + bc1_ref[...], 0.0)


def _stage1(x, W1, b1, Wc1, bc1):
    return pl.pallas_call(
        _stage1_body,
        grid=(NB,),
        in_specs=[
            pl.BlockSpec((BN, D), lambda i: (i, 0)),
            pl.BlockSpec((D, H), lambda i: (0, 0)),
            pl.BlockSpec((1, H), lambda i: (0, 0)),
            pl.BlockSpec((D, H), lambda i: (0, 0)),
            pl.BlockSpec((1, H), lambda i: (0, 0)),
        ],
        out_specs=[
            pl.BlockSpec((BN, H), lambda i: (i, 0)),
            pl.BlockSpec((BN, H), lambda i: (i, 0)),
        ],
        out_shape=[
            jax.ShapeDtypeStruct((N, H), jnp.float32),
            jax.ShapeDtypeStruct((N, H), jnp.float32),
        ],
    )(x, W1, b1.reshape(1, H), Wc1, bc1.reshape(1, H))


def _segsum_sc(h, src3, dst3, zeros):
    @functools.partial(
        pl.kernel,
        out_type=jax.ShapeDtypeStruct((NC, NP, H), jnp.float32),
        mesh=_sc_mesh(),
        compiler_params=pltpu.CompilerParams(needs_layout_passes=False),
        scratch_types=[
            pltpu.VMEM((JB, C2), jnp.int32),
            pltpu.VMEM((JB, C2), jnp.int32),
            pltpu.VMEM((C2, H), jnp.float32),
            pltpu.VMEM((C2, H), jnp.float32),
            pltpu.VMEM_SHARED((NP, H), jnp.float32),
            pltpu.SemaphoreType.DMA,
            pltpu.SemaphoreType.DMA,
        ],
    )
    def k(h_hbm, src_hbm, dst_hbm, z_hbm, out_hbm, srcb, dstb, rows0, rows1,
          acc_sh, sem0, sem1):
        c = lax.axis_index("c")
        s = lax.axis_index("s")
        wid = c * NS + s
        pltpu.sync_copy(z_hbm.at[pl.ds(s * TPS, TPS)],
                        acc_sh.at[pl.ds(s * TPS, TPS)])
        plsc.subcore_barrier()
        bufs = (rows0, rows1)
        sems = (sem0, sem1)

        def blk(jb, carry):
            base = jb * JB
            pltpu.sync_copy(src_hbm.at[wid, pl.ds(base, JB)], srcb)
            pltpu.sync_copy(dst_hbm.at[wid, pl.ds(base, JB)], dstb)
            pltpu.async_copy(h_hbm.at[srcb.at[0]], rows0, sem0)
            for jj in range(JB):
                b = bufs[jj % 2]
                sm = sems[jj % 2]
                if jj + 1 < JB:
                    pltpu.async_copy(h_hbm.at[srcb.at[jj + 1]],
                                     bufs[(jj + 1) % 2], sems[(jj + 1) % 2])
                pltpu.make_async_copy(h_hbm.at[srcb.at[jj]], b, sm).wait()
                pltpu.sync_copy(b, acc_sh.at[dstb.at[jj]], add=True)
            return carry

        lax.fori_loop(0, NJB2, blk, 0)
        plsc.subcore_barrier()
        pltpu.sync_copy(acc_sh.at[pl.ds(s * TPS, TPS)],
                        out_hbm.at[c, pl.ds(s * TPS, TPS)])

    return k(h, src3, dst3, zeros)


def _stage3_body(h_ref, m0_ref, m1_ref, W2_ref, b2_ref, Was_ref,
                 Wad_ref, ba1_ref, pt_ref, qt_ref):
    emb = jnp.maximum(
        jnp.dot(h_ref[...] + m0_ref[...] + m1_ref[...], W2_ref[...],
                preferred_element_type=jnp.float32) + b2_ref[...], 0.0)
    p = (jnp.dot(emb, Was_ref[...], preferred_element_type=jnp.float32)
         + ba1_ref[...])
    q = jnp.dot(emb, Wad_ref[...], preferred_element_type=jnp.float32)
    pad = jnp.zeros((BN, H - A), jnp.float32)
    pt_ref[...] = jnp.concatenate([p, pad], axis=1)
    qt_ref[...] = jnp.concatenate([q, pad], axis=1)


def _stage3(h, m0, m1, W2, b2, Was, Wad, ba1):
    return pl.pallas_call(
        _stage3_body,
        grid=(NB,),
        in_specs=[
            pl.BlockSpec((BN, H), lambda i: (i, 0)),
            pl.BlockSpec((BN, H), lambda i: (i, 0)),
            pl.BlockSpec((BN, H), lambda i: (i, 0)),
            pl.BlockSpec((H, H), lambda i: (0, 0)),
            pl.BlockSpec((1, H), lambda i: (0, 0)),
            pl.BlockSpec((H, A), lambda i: (0, 0)),
            pl.BlockSpec((H, A), lambda i: (0, 0)),
            pl.BlockSpec((1, A), lambda i: (0, 0)),
        ],
        out_specs=[
            pl.BlockSpec((BN, H), lambda i: (i, 0)),
            pl.BlockSpec((BN, H), lambda i: (i, 0)),
        ],
        out_shape=[
            jax.ShapeDtypeStruct((N, H), jnp.float32),
            jax.ShapeDtypeStruct((N, H), jnp.float32),
        ],
    )(h, m0, m1, W2, b2.reshape(1, H), Was, Wad, ba1.reshape(1, A))


def _edge_att_sc(pt, qt, g0, src3, dst3, zeros, wx):
    @functools.partial(
        pl.kernel,
        out_type=[
            jax.ShapeDtypeStruct((NW, NCH4, H), jnp.float32),
            jax.ShapeDtypeStruct((NC, NP, H), jnp.float32),
        ],
        mesh=_sc_mesh(),
        compiler_params=pltpu.CompilerParams(needs_layout_passes=False),
        scratch_types=[
            pltpu.VMEM((JB, C4), jnp.int32),
            pltpu.VMEM((JB, C4), jnp.int32),
            pltpu.VMEM((CP4, H), jnp.float32),
            pltpu.VMEM((CP4, H), jnp.float32),
            pltpu.VMEM((CP4, H), jnp.float32),
            pltpu.VMEM((CP4, H), jnp.float32),
            pltpu.VMEM((CP4, H), jnp.float32),
            pltpu.VMEM((CP4, H), jnp.float32),
            pltpu.VMEM((JB, H), jnp.float32),
            pltpu.VMEM((8, 16), jnp.float32),
            pltpu.VMEM_SHARED((NP, H), jnp.float32),
            pltpu.SemaphoreType.DMA,
            pltpu.SemaphoreType.DMA,
            pltpu.SemaphoreType.DMA,
            pltpu.SemaphoreType.DMA,
            pltpu.SemaphoreType.DMA,
            pltpu.SemaphoreType.DMA,
        ],
    )
    def k(pt_hbm, qt_hbm, g_hbm, src_hbm, dst_hbm, z_hbm, wx_hbm,
          att_hbm, m2_hbm,
          srcb, dstb, pt0, pt1, qt0, qt1, gv0, gv1, attb, wx_v, acc_sh,
          psem0, psem1, qsem0, qsem1, gsem0, gsem1):
        c = lax.axis_index("c")
        s = lax.axis_index("s")
        wid = c * NS + s
        pltpu.sync_copy(z_hbm.at[pl.ds(s * TPS, TPS)],
                        acc_sh.at[pl.ds(s * TPS, TPS)])
        pltpu.sync_copy(wx_hbm, wx_v)
        zv = jnp.zeros((16,), jnp.float32)
        for e in range(C4, CP4):
            for i in range(H // 16):
                for buf in (pt0, pt1, qt0, qt1, gv0, gv1):
                    buf[e, pl.ds(i * 16, 16)] = zv
        plsc.subcore_barrier()

        pts = (pt0, pt1)
        qts = (qt0, qt1)
        gvs = (gv0, gv1)
        psems = (psem0, psem1)
        qsems = (qsem0, qsem1)
        gsems = (gsem0, gsem1)

        def start(jj, ph):
            pltpu.async_copy(pt_hbm.at[srcb.at[jj]],
                             pts[ph].at[pl.ds(0, C4)], psems[ph])
            pltpu.async_copy(qt_hbm.at[dstb.at[jj]],
                             qts[ph].at[pl.ds(0, C4)], qsems[ph])
            pltpu.async_copy(g_hbm.at[srcb.at[jj]],
                             gvs[ph].at[pl.ds(0, C4)], gsems[ph])

        def wait(jj, ph):
            pltpu.make_async_copy(pt_hbm.at[srcb.at[jj]],
                                  pts[ph].at[pl.ds(0, C4)], psems[ph]).wait()
            pltpu.make_async_copy(qt_hbm.at[dstb.at[jj]],
                                  qts[ph].at[pl.ds(0, C4)], qsems[ph]).wait()
            pltpu.make_async_copy(g_hbm.at[srcb.at[jj]],
                                  gvs[ph].at[pl.ds(0, C4)], gsems[ph]).wait()

        def compute(jj, ph):
            pt_v = pts[ph]
            qt_v = qts[ph]
            g_v = gvs[ph]
            w0 = wx_v[0]
            w1 = wx_v[1]
            w2 = wx_v[2]
            w3 = wx_v[3]

            def grp(gi, cc2):
                acc = wx_v[4]
                for l in range(16):
                    e = gi * 16 + l
                    a0 = jnp.maximum(
                        pt_v[e, pl.ds(0, 16)] + qt_v[e, pl.ds(0, 16)], 0.0)
                    a1 = jnp.maximum(
                        pt_v[e, pl.ds(16, 16)] + qt_v[e, pl.ds(16, 16)], 0.0)
                    a2 = jnp.maximum(
                        pt_v[e, pl.ds(32, 16)] + qt_v[e, pl.ds(32, 16)], 0.0)
                    a3 = jnp.maximum(
                        pt_v[e, pl.ds(48, 16)] + qt_v[e, pl.ds(48, 16)], 0.0)
                    t = (_round_bf16(a0) * w0 + _round_bf16(a1) * w1
                         + _round_bf16(a2) * w2 + _round_bf16(a3) * w3)
                    sv = jnp.sum(t)
                    acc = jnp.where(lax.iota(jnp.int32, 16) == l, sv, acc)
                attvec = 1.0 / (1.0 + jnp.exp(-acc))
                attb[jj, pl.ds(gi * 16, 16)] = attvec
                for l in range(16):
                    av = attvec[l]
                    e = gi * 16 + l
                    for i in range(H // 16):
                        g_v[e, pl.ds(i * 16, 16)] = (
                            av * g_v[e, pl.ds(i * 16, 16)])
                return cc2

            lax.fori_loop(0, CP4 // 16, grp, 0)
            pltpu.sync_copy(g_v.at[pl.ds(0, C4)],
                            acc_sh.at[dstb.at[jj]], add=True)

        def blk(jb, carry):
            base = jb * JB
            pltpu.sync_copy(src_hbm.at[wid, pl.ds(base, JB)], srcb)
            pltpu.sync_copy(dst_hbm.at[wid, pl.ds(base, JB)], dstb)
            start(0, 0)
            for jj in range(JB):
                if jj + 1 < JB:
                    start(jj + 1, (jj + 1) % 2)
                wait(jj, jj % 2)
                compute(jj, jj % 2)
            pltpu.sync_copy(attb, att_hbm.at[wid, pl.ds(base, JB)])
            return carry

        lax.fori_loop(0, NJB4, blk, 0)
        plsc.subcore_barrier()
        pltpu.sync_copy(acc_sh.at[pl.ds(s * TPS, TPS)],
                        m2_hbm.at[c, pl.ds(s * TPS, TPS)])

    return k(pt, qt, g0, src3, dst3, zeros, wx)


def _stage5_body(g_ref, m0_ref, m1_ref, Wc2_ref, bc2_ref, batch_ref, Wo_ref,
                 bo_ref, out_ref, pool_acc, cnt_acc):
    i = pl.program_id(0)
    gf = jnp.maximum(
        jnp.dot(g_ref[...] + m0_ref[...] + m1_ref[...], Wc2_ref[...],
                preferred_element_type=jnp.float32) + bc2_ref[...], 0.0)
    b = batch_ref[0, 0, :]
    onehot = (b[:, None] == lax.broadcasted_iota(jnp.int32, (BN, G), 1)
              ).astype(jnp.float32)
    pool_p = lax.dot_general(onehot, gf, (((0,), (0,)), ((), ())),
                             preferred_element_type=jnp.float32,
                             precision=lax.Precision.HIGHEST)
    cnt_p = lax.dot_general(onehot, jnp.ones((BN, 1), jnp.float32),
                            (((0,), (0,)), ((), ())),
                            preferred_element_type=jnp.float32,
                            precision=lax.Precision.HIGHEST)

    @pl.when(i == 0)
    def _():
        pool_acc[...] = jnp.zeros_like(pool_acc)
        cnt_acc[...] = jnp.zeros_like(cnt_acc)

    pool_acc[...] += pool_p
    cnt_acc[...] += cnt_p

    @pl.when(i == NB - 1)
    def _():
        pooled = pool_acc[...] / jnp.maximum(cnt_acc[...], 1.0)
        out_ref[...] = (jnp.dot(pooled, Wo_ref[...],
                                preferred_element_type=jnp.float32)
                        + bo_ref[...])


def _stage5(g0, m0, m1, Wc2, bc2, batch3, Wo, bo):
    return pl.pallas_call(
        _stage5_body,
        grid=(NB,),
        in_specs=[
            pl.BlockSpec((BN, H), lambda i: (i, 0)),
            pl.BlockSpec((BN, H), lambda i: (i, 0)),
            pl.BlockSpec((BN, H), lambda i: (i, 0)),
            pl.BlockSpec((H, H), lambda i: (0, 0)),
            pl.BlockSpec((1, H), lambda i: (0, 0)),
            pl.BlockSpec((1, 1, BN), lambda i: (i, 0, 0)),
            pl.BlockSpec((H, C), lambda i: (0, 0)),
            pl.BlockSpec((1, C), lambda i: (0, 0)),
        ],
        out_specs=pl.BlockSpec((G, C), lambda i: (0, 0)),
        out_shape=jax.ShapeDtypeStruct((G, C), jnp.float32),
        scratch_shapes=[
            pltpu.VMEM((G, H), jnp.float32),
            pltpu.VMEM((G, 1), jnp.float32),
        ],
        compiler_params=pltpu.CompilerParams(
            dimension_semantics=("arbitrary",)),
    )(g0, m0, m1, Wc2, bc2.reshape(1, H), batch3, Wo, bo.reshape(1, C))


def kernel(x, W1, b1, W2, b2, Wa1, ba1, Wa2, ba2, Wc1, bc1, Wc2, bc2, Wo, bo,
           edge_index, batch):
    src2 = edge_index[0].reshape(NW, NCH2, C2)
    dst2 = edge_index[1].reshape(NW, NCH2, C2)
    src4 = edge_index[0].reshape(NW, NCH4, C4)
    dst4 = edge_index[1].reshape(NW, NCH4, C4)
    zeros = jnp.zeros((NP, H), jnp.float32)
    wx = jnp.zeros((8, 16), jnp.float32)
    wa2_b = Wa2[:, 0].astype(jnp.bfloat16).astype(jnp.float32)
    wx = wx.at[:4].set(wa2_b.reshape(4, 16))
    wx = wx.at[4].set(ba2[0])

    h, g0 = _stage1(x, W1, b1, Wc1, bc1)
    msum = _segsum_sc(h, src2, dst2, zeros)
    pt, qt = _stage3(h, msum[0, :N], msum[1, :N], W2, b2,
                     Wa1[:H], Wa1[H:], ba1)
    att3, m2 = _edge_att_sc(pt, qt, g0, src4, dst4, zeros, wx)
    logits = _stage5(g0, m2[0, :N], m2[1, :N], Wc2, bc2,
                     batch.reshape(NB, 1, BN), Wo, bo)
    return att3[:, :, :C4].reshape(-1), logits


# speedup vs baseline: 5.2363x; 1.0195x over previous
import functools

import jax
import jax.numpy as jnp
from jax import lax
from jax.experimental import pallas as pl
from jax.experimental.pallas import tpu as pltpu
from jax.experimental.pallas import tpu_sc as plsc

N = 10000
E = 320000
D = 128
H = 128
A = 64
G = 64
C = 3

NC = 2
NS = 16
NW = NC * NS
EPW = E // NW
JB = 8

C2 = 125
NCH2 = EPW // C2
NJB2 = NCH2 // JB

C4 = 25
CP4 = 32
NCH4 = EPW // C4
NJB4 = NCH4 // JB

NP = 10240
TPS = NP // NS
BN = 2000
NB = N // BN


def _round_bf16(v):
    u = lax.bitcast_convert_type(v, jnp.int32)
    lsb = lax.shift_right_logical(u, 16) & 1
    r = u + 32767 + lsb
    r = r & jnp.int32(-65536)
    return lax.bitcast_convert_type(r, jnp.float32)


def _sc_mesh():
    return plsc.VectorSubcoreMesh(
        core_axis_name="c", subcore_axis_name="s",
        num_cores=NC, num_subcores=NS)


def _stage1_body(x_ref, W1_ref, b1_ref, Wc1_ref, bc1_ref, h_ref, g_ref):
    xb = x_ref[...]
    h_ref[...] = jnp.maximum(
        jnp.dot(xb, W1_ref[...], preferred_element_type=jnp.float32)
        + b1_ref[...], 0.0)
    g_ref[...] = jnp.maximum(
        jnp.dot(xb, Wc1_ref[...], preferred_element_type=jnp.float32)
        + bc1_ref[...], 0.0)


def _stage1(x, W1, b1, Wc1, bc1):
    return pl.pallas_call(
        _stage1_body,
        grid=(NB,),
        in_specs=[
            pl.BlockSpec((BN, D), lambda i: (i, 0)),
            pl.BlockSpec((D, H), lambda i: (0, 0)),
            pl.BlockSpec((1, H), lambda i: (0, 0)),
            pl.BlockSpec((D, H), lambda i: (0, 0)),
            pl.BlockSpec((1, H), lambda i: (0, 0)),
        ],
        out_specs=[
            pl.BlockSpec((BN, H), lambda i: (i, 0)),
            pl.BlockSpec((BN, H), lambda i: (i, 0)),
        ],
        out_shape=[
            jax.ShapeDtypeStruct((N, H), jnp.float32),
            jax.ShapeDtypeStruct((N, H), jnp.float32),
        ],
    )(x, W1, b1.reshape(1, H), Wc1, bc1.reshape(1, H))


def _segsum_sc(h, src3, dst3, zeros):
    @functools.partial(
        pl.kernel,
        out_type=jax.ShapeDtypeStruct((NC, NP, H), jnp.float32),
        mesh=_sc_mesh(),
        compiler_params=pltpu.CompilerParams(needs_layout_passes=False),
        scratch_types=[
            pltpu.VMEM((JB, C2), jnp.int32),
            pltpu.VMEM((JB, C2), jnp.int32),
            pltpu.VMEM((C2, H), jnp.float32),
            pltpu.VMEM((C2, H), jnp.float32),
            pltpu.VMEM_SHARED((NP, H), jnp.float32),
            pltpu.SemaphoreType.DMA,
            pltpu.SemaphoreType.DMA,
        ],
    )
    def k(h_hbm, src_hbm, dst_hbm, z_hbm, out_hbm, srcb, dstb, rows0, rows1,
          acc_sh, sem0, sem1):
        c = lax.axis_index("c")
        s = lax.axis_index("s")
        wid = c * NS + s
        pltpu.sync_copy(z_hbm.at[pl.ds(s * TPS, TPS)],
                        acc_sh.at[pl.ds(s * TPS, TPS)])
        plsc.subcore_barrier()
        bufs = (rows0, rows1)
        sems = (sem0, sem1)

        def blk(jb, carry):
            base = jb * JB
            pltpu.sync_copy(src_hbm.at[wid, pl.ds(base, JB)], srcb)
            pltpu.sync_copy(dst_hbm.at[wid, pl.ds(base, JB)], dstb)
            pltpu.async_copy(h_hbm.at[srcb.at[0]], rows0, sem0)
            for jj in range(JB):
                b = bufs[jj % 2]
                sm = sems[jj % 2]
                if jj + 1 < JB:
                    pltpu.async_copy(h_hbm.at[srcb.at[jj + 1]],
                                     bufs[(jj + 1) % 2], sems[(jj + 1) % 2])
                pltpu.make_async_copy(h_hbm.at[srcb.at[jj]], b, sm).wait()
                pltpu.sync_copy(b, acc_sh.at[dstb.at[jj]], add=True)
            return carry

        lax.fori_loop(0, NJB2, blk, 0)
        plsc.subcore_barrier()
        pltpu.sync_copy(acc_sh.at[pl.ds(s * TPS, TPS)],
                        out_hbm.at[c, pl.ds(s * TPS, TPS)])

    return k(h, src3, dst3, zeros)


def _stage3_body(h_ref, m_ref, W2_ref, b2_ref, Was_ref,
                 Wad_ref, ba1_ref, pt_ref, qt_ref):
    emb = jnp.maximum(
        jnp.dot(h_ref[...] + m_ref[0] + m_ref[1], W2_ref[...],
                preferred_element_type=jnp.float32) + b2_ref[...], 0.0)
    p = (jnp.dot(emb, Was_ref[...], preferred_element_type=jnp.float32)
         + ba1_ref[...])
    q = jnp.dot(emb, Wad_ref[...], preferred_element_type=jnp.float32)
    pad = jnp.zeros((BN, H - A), jnp.float32)
    pt_ref[...] = jnp.concatenate([p, pad], axis=1)
    qt_ref[...] = jnp.concatenate([q, pad], axis=1)


def _stage3(h, msum, W2, b2, Was, Wad, ba1):
    return pl.pallas_call(
        _stage3_body,
        grid=(NB,),
        in_specs=[
            pl.BlockSpec((BN, H), lambda i: (i, 0)),
            pl.BlockSpec((NC, BN, H), lambda i: (0, i, 0)),
            pl.BlockSpec((H, H), lambda i: (0, 0)),
            pl.BlockSpec((1, H), lambda i: (0, 0)),
            pl.BlockSpec((H, A), lambda i: (0, 0)),
            pl.BlockSpec((H, A), lambda i: (0, 0)),
            pl.BlockSpec((1, A), lambda i: (0, 0)),
        ],
        out_specs=[
            pl.BlockSpec((BN, H), lambda i: (i, 0)),
            pl.BlockSpec((BN, H), lambda i: (i, 0)),
        ],
        out_shape=[
            jax.ShapeDtypeStruct((N, H), jnp.float32),
            jax.ShapeDtypeStruct((N, H), jnp.float32),
        ],
    )(h, msum, W2, b2.reshape(1, H), Was, Wad, ba1.reshape(1, A))


def _edge_att_sc(pt, qt, g0, src3, dst3, zeros, wx):
    @functools.partial(
        pl.kernel,
        out_type=[
            jax.ShapeDtypeStruct((NW, NCH4, H), jnp.float32),
            jax.ShapeDtypeStruct((NC, NP, H), jnp.float32),
        ],
        mesh=_sc_mesh(),
        compiler_params=pltpu.CompilerParams(needs_layout_passes=False),
        scratch_types=[
            pltpu.VMEM((JB, C4), jnp.int32),
            pltpu.VMEM((JB, C4), jnp.int32),
            pltpu.VMEM((CP4, H), jnp.float32),
            pltpu.VMEM((CP4, H), jnp.float32),
            pltpu.VMEM((CP4, H), jnp.float32),
            pltpu.VMEM((CP4, H), jnp.float32),
            pltpu.VMEM((CP4, H), jnp.float32),
            pltpu.VMEM((CP4, H), jnp.float32),
            pltpu.VMEM((JB, H), jnp.float32),
            pltpu.VMEM((8, 16), jnp.float32),
            pltpu.VMEM_SHARED((NP, H), jnp.float32),
            pltpu.SemaphoreType.DMA,
            pltpu.SemaphoreType.DMA,
            pltpu.SemaphoreType.DMA,
            pltpu.SemaphoreType.DMA,
            pltpu.SemaphoreType.DMA,
            pltpu.SemaphoreType.DMA,
        ],
    )
    def k(pt_hbm, qt_hbm, g_hbm, src_hbm, dst_hbm, z_hbm, wx_hbm,
          att_hbm, m2_hbm,
          srcb, dstb, pt0, pt1, qt0, qt1, gv0, gv1, attb, wx_v, acc_sh,
          psem0, psem1, qsem0, qsem1, gsem0, gsem1):
        c = lax.axis_index("c")
        s = lax.axis_index("s")
        wid = c * NS + s
        pltpu.sync_copy(z_hbm.at[pl.ds(s * TPS, TPS)],
                        acc_sh.at[pl.ds(s * TPS, TPS)])
        pltpu.sync_copy(wx_hbm, wx_v)
        zv = jnp.zeros((16,), jnp.float32)
        for e in range(C4, CP4):
            for i in range(H // 16):
                for buf in (pt0, pt1, qt0, qt1, gv0, gv1):
                    buf[e, pl.ds(i * 16, 16)] = zv
        plsc.subcore_barrier()

        pts = (pt0, pt1)
        qts = (qt0, qt1)
        gvs = (gv0, gv1)
        psems = (psem0, psem1)
        qsems = (qsem0, qsem1)
        gsems = (gsem0, gsem1)

        def start(jj, ph):
            pltpu.async_copy(pt_hbm.at[srcb.at[jj]],
                             pts[ph].at[pl.ds(0, C4)], psems[ph])
            pltpu.async_copy(qt_hbm.at[dstb.at[jj]],
                             qts[ph].at[pl.ds(0, C4)], qsems[ph])
            pltpu.async_copy(g_hbm.at[srcb.at[jj]],
                             gvs[ph].at[pl.ds(0, C4)], gsems[ph])

        def wait(jj, ph):
            pltpu.make_async_copy(pt_hbm.at[srcb.at[jj]],
                                  pts[ph].at[pl.ds(0, C4)], psems[ph]).wait()
            pltpu.make_async_copy(qt_hbm.at[dstb.at[jj]],
                                  qts[ph].at[pl.ds(0, C4)], qsems[ph]).wait()
            pltpu.make_async_copy(g_hbm.at[srcb.at[jj]],
                                  gvs[ph].at[pl.ds(0, C4)], gsems[ph]).wait()

        def compute(jj, ph):
            pt_v = pts[ph]
            qt_v = qts[ph]
            g_v = gvs[ph]
            w0 = wx_v[0]
            w1 = wx_v[1]
            w2 = wx_v[2]
            w3 = wx_v[3]

            def grp(gi, cc2):
                acc = wx_v[4]
                for l in range(16):
                    e = gi * 16 + l
                    a0 = jnp.maximum(
                        pt_v[e, pl.ds(0, 16)] + qt_v[e, pl.ds(0, 16)], 0.0)
                    a1 = jnp.maximum(
                        pt_v[e, pl.ds(16, 16)] + qt_v[e, pl.ds(16, 16)], 0.0)
                    a2 = jnp.maximum(
                        pt_v[e, pl.ds(32, 16)] + qt_v[e, pl.ds(32, 16)], 0.0)
                    a3 = jnp.maximum(
                        pt_v[e, pl.ds(48, 16)] + qt_v[e, pl.ds(48, 16)], 0.0)
                    t = (_round_bf16(a0) * w0 + _round_bf16(a1) * w1
                         + _round_bf16(a2) * w2 + _round_bf16(a3) * w3)
                    sv = jnp.sum(t)
                    acc = jnp.where(lax.iota(jnp.int32, 16) == l, sv, acc)
                attvec = 1.0 / (1.0 + jnp.exp(-acc))
                attb[jj, pl.ds(gi * 16, 16)] = attvec
                for l in range(16):
                    av = attvec[l]
                    e = gi * 16 + l
                    for i in range(H // 16):
                        g_v[e, pl.ds(i * 16, 16)] = (
                            av * g_v[e, pl.ds(i * 16, 16)])
                return cc2

            lax.fori_loop(0, CP4 // 16, grp, 0)
            pltpu.sync_copy(g_v.at[pl.ds(0, C4)],
                            acc_sh.at[dstb.at[jj]], add=True)

        def blk(jb, carry):
            base = jb * JB
            pltpu.sync_copy(src_hbm.at[wid, pl.ds(base, JB)], srcb)
            pltpu.sync_copy(dst_hbm.at[wid, pl.ds(base, JB)], dstb)
            start(0, 0)
            for jj in range(JB):
                if jj + 1 < JB:
                    start(jj + 1, (jj + 1) % 2)
                wait(jj, jj % 2)
                compute(jj, jj % 2)
            pltpu.sync_copy(attb, att_hbm.at[wid, pl.ds(base, JB)])
            return carry

        lax.fori_loop(0, NJB4, blk, 0)
        plsc.subcore_barrier()
        pltpu.sync_copy(acc_sh.at[pl.ds(s * TPS, TPS)],
                        m2_hbm.at[c, pl.ds(s * TPS, TPS)])

    return k(pt, qt, g0, src3, dst3, zeros, wx)


def _stage5_body(g_ref, m_ref, Wc2_ref, bc2_ref, batch_ref, Wo_ref,
                 bo_ref, out_ref, pool_acc, cnt_acc):
    i = pl.program_id(0)
    gf = jnp.maximum(
        jnp.dot(g_ref[...] + m_ref[0] + m_ref[1], Wc2_ref[...],
                preferred_element_type=jnp.float32) + bc2_ref[...], 0.0)
    b = batch_ref[0, 0, :]
    onehot = (b[:, None] == lax.broadcasted_iota(jnp.int32, (BN, G), 1)
              ).astype(jnp.float32)
    pool_p = lax.dot_general(onehot, gf, (((0,), (0,)), ((), ())),
                             preferred_element_type=jnp.float32,
                             precision=lax.Precision.HIGHEST)
    cnt_p = lax.dot_general(onehot, jnp.ones((BN, 1), jnp.float32),
                            (((0,), (0,)), ((), ())),
                            preferred_element_type=jnp.float32,
                            precision=lax.Precision.HIGHEST)

    @pl.when(i == 0)
    def _():
        pool_acc[...] = jnp.zeros_like(pool_acc)
        cnt_acc[...] = jnp.zeros_like(cnt_acc)

    pool_acc[...] += pool_p
    cnt_acc[...] += cnt_p

    @pl.when(i == NB - 1)
    def _():
        pooled = pool_acc[...] / jnp.maximum(cnt_acc[...], 1.0)
        out_ref[...] = (jnp.dot(pooled, Wo_ref[...],
                                preferred_element_type=jnp.float32)
                        + bo_ref[...])


def _stage5(g0, m2, Wc2, bc2, batch3, Wo, bo):
    return pl.pallas_call(
        _stage5_body,
        grid=(NB,),
        in_specs=[
            pl.BlockSpec((BN, H), lambda i: (i, 0)),
            pl.BlockSpec((NC, BN, H), lambda i: (0, i, 0)),
            pl.BlockSpec((H, H), lambda i: (0, 0)),
            pl.BlockSpec((1, H), lambda i: (0, 0)),
            pl.BlockSpec((1, 1, BN), lambda i: (i, 0, 0)),
            pl.BlockSpec((H, C), lambda i: (0, 0)),
            pl.BlockSpec((1, C), lambda i: (0, 0)),
        ],
        out_specs=pl.BlockSpec((G, C), lambda i: (0, 0)),
        out_shape=jax.ShapeDtypeStruct((G, C), jnp.float32),
        scratch_shapes=[
            pltpu.VMEM((G, H), jnp.float32),
            pltpu.VMEM((G, 1), jnp.float32),
        ],
        compiler_params=pltpu.CompilerParams(
            dimension_semantics=("arbitrary",)),
    )(g0, m2, Wc2, bc2.reshape(1, H), batch3, Wo, bo.reshape(1, C))


def kernel(x, W1, b1, W2, b2, Wa1, ba1, Wa2, ba2, Wc1, bc1, Wc2, bc2, Wo, bo,
           edge_index, batch):
    src2 = edge_index[0].reshape(NW, NCH2, C2)
    dst2 = edge_index[1].reshape(NW, NCH2, C2)
    src4 = edge_index[0].reshape(NW, NCH4, C4)
    dst4 = edge_index[1].reshape(NW, NCH4, C4)
    zeros = jnp.zeros((NP, H), jnp.float32)
    wx = jnp.zeros((8, 16), jnp.float32)
    wa2_b = Wa2[:, 0].astype(jnp.bfloat16).astype(jnp.float32)
    wx = wx.at[:4].set(wa2_b.reshape(4, 16))
    wx = wx.at[4].set(ba2[0])

    h, g0 = _stage1(x, W1, b1, Wc1, bc1)
    msum = _segsum_sc(h, src2, dst2, zeros)
    pt, qt = _stage3(h, msum, W2, b2, Wa1[:H], Wa1[H:], ba1)
    att3, m2 = _edge_att_sc(pt, qt, g0, src4, dst4, zeros, wx)
    logits = _stage5(g0, m2, Wc2, bc2, batch.reshape(NB, 1, BN), Wo, bo)
    return att3[:, :, :C4].reshape(-1), logits
